# baseline (speedup 1.0000x reference)
import functools
import math

import jax
import jax.numpy as jnp
from jax import lax
from jax.experimental import pallas as pl
from jax.experimental.pallas import tpu as pltpu
from jax.experimental.pallas import tpu_sc as plsc

N_NODES = 10000
N_EDGES = 320000
D_FEAT = 128
CED = 16
MLP_H = 32
EPS = 1e-06
P = 6

NN_PAD = 10240
NW = 32
E_PER_W = N_EDGES // NW
NODES_PER_TILE = NN_PAD // 16

E1 = 32000
E2 = 64000

_CC = (((0,), (1,)), ((), ()))


def _pass1_body(x_ref, wx1a_ref, wx2_ref, wx3_ref, wc1_ref, wc2_ref,
                at_ref, chis_ref):
    x = x_ref[...]
    w_edge = (wx1a_ref[...] @ wx2_ref[...]) @ wx3_ref[...]
    w_edge = w_edge * (1.0 / (12.0 * 32.0))
    at_ref[...] = lax.dot_general(w_edge, x, _CC,
                                  preferred_element_type=jnp.float32
                                  ).astype(jnp.bfloat16)
    wct = lax.dot_general(wc2_ref[...], wc1_ref[...], _CC)
    wct = wct * (1.0 / math.sqrt(128.0 * 16.0))
    chis = lax.dot_general(wct, x, (((1,), (1,)), ((), ())),
                           preferred_element_type=jnp.float32)
    chis_ref[...] = chis.reshape(1, 1, E1)


def _pass1(x, wx1a, wx2, wx3, wc1, wc2):
    nb = N_EDGES // E1
    return pl.pallas_call(
        _pass1_body,
        grid=(nb,),
        in_specs=[
            pl.BlockSpec((E1, D_FEAT), lambda i: (i, 0)),
            pl.BlockSpec((D_FEAT, MLP_H), lambda i: (0, 0)),
            pl.BlockSpec((MLP_H, MLP_H), lambda i: (0, 0)),
            pl.BlockSpec((MLP_H, MLP_H), lambda i: (0, 0)),
            pl.BlockSpec((D_FEAT, CED), lambda i: (0, 0)),
            pl.BlockSpec((CED, 1), lambda i: (0, 0)),
        ],
        out_specs=[
            pl.BlockSpec((MLP_H, E1), lambda i: (0, i)),
            pl.BlockSpec((1, 1, E1), lambda i: (i, 0, 0)),
        ],
        out_shape=[
            jax.ShapeDtypeStruct((MLP_H, N_EDGES), jnp.bfloat16),
            jax.ShapeDtypeStruct((N_EDGES // E1, 1, E1), jnp.float32),
        ],
    )(x, wx1a, wx2, wx3, wc1, wc2)


CHUNK = 2000
CH_PER_W = E_PER_W // CHUNK
CH_PER_ROW1 = E1 // CHUNK
CH_PER_ROW2 = E2 // CHUNK


def _sc_scatter_body(sends_hbm, vals_hbm, out_hbm, idx_v, val_v, zero_v, acc_sh):
    cid = lax.axis_index("c")
    sid = lax.axis_index("s")
    z16 = jnp.zeros((16,), jnp.float32)

    def _zb(i, carry):
        zero_v[pl.ds(i * 16, 16)] = z16
        return carry

    lax.fori_loop(0, NODES_PER_TILE // 16, _zb, 0)
    pltpu.sync_copy(zero_v, acc_sh.at[pl.ds(sid * NODES_PER_TILE, NODES_PER_TILE)])
    plsc.subcore_barrier()

    base = (cid * 16 + sid) * E_PER_W
    pltpu.sync_copy(sends_hbm.at[pl.ds(base, E_PER_W)], idx_v)
    pltpu.sync_copy(vals_hbm.at[pl.ds(base, E_PER_W)], val_v)
    pltpu.sync_copy(val_v, acc_sh.at[idx_v], add=True)
    plsc.subcore_barrier()

    pltpu.sync_copy(acc_sh.at[pl.ds(sid * NODES_PER_TILE, NODES_PER_TILE)],
                    out_hbm.at[cid, pl.ds(sid * NODES_PER_TILE, NODES_PER_TILE)])


def _sc_scatter(senders, chis_flat):
    mesh = plsc.VectorSubcoreMesh(core_axis_name="c", subcore_axis_name="s")
    f = functools.partial(
        pl.kernel,
        out_type=jax.ShapeDtypeStruct((2, NN_PAD), jnp.float32),
        mesh=mesh,
        scratch_types=[
            pltpu.VMEM((E_PER_W,), jnp.int32),
            pltpu.VMEM((E_PER_W,), jnp.float32),
            pltpu.VMEM((NODES_PER_TILE,), jnp.float32),
            pltpu.VMEM_SHARED((NN_PAD,), jnp.float32),
        ],
    )(_sc_scatter_body)
    return f(senders, chis_flat)


def _node_body(p_ref, spe_ref, rad_ref, hard_ref, ce_ref,
               ww1a_ref, ww1b_ref, wx1b_ref, wx2_ref, wx3_ref,
               charges_ref, pot_ref, mean_ref, emb2_ref, wnw0_ref, diags_ref):
    ones21 = jnp.ones((2, 1), jnp.float32)
    chis = lax.dot_general(p_ref[...], ones21, (((0,), (0,)), ((), ())),
                           preferred_element_type=jnp.float32)
    spe = spe_ref[...]
    lane = lax.broadcasted_iota(jnp.int32, (1, 128), 1)
    oh = (spe == lane).astype(jnp.float32)

    hard = hard_ref[...]
    sp = jnp.maximum(hard, 0.0) + jnp.log1p(jnp.exp(-jnp.abs(hard)))
    gam = rad_ref[...] * 4.0 + 0.5
    diag_s = sp + 2.0 / (gam * math.sqrt(math.pi))

    diag = lax.dot_general(oh, diag_s, (((1,), (1,)), ((), ())),
                           preferred_element_type=jnp.float32)
    charges_raw = -chis / (diag + EPS)
    ridx = lax.broadcasted_iota(jnp.int32, (NN_PAD, 1), 0)
    mask = (ridx < N_NODES).astype(jnp.float32)
    charges_raw = charges_raw * mask
    mean = jnp.sum(charges_raw) / N_NODES
    charges = charges_raw - mean
    charges_ref[...] = charges
    pot_ref[...] = jnp.sum(
        mask * (chis * charges + 0.5 * diag * charges * charges)
    ).reshape(1, 1)
    mean_ref[...] = mean.reshape(1, 1)
    diags_ref[...] = diag_s
    node_mat = (wx1b_ref[...] @ wx2_ref[...]) @ wx3_ref[...]
    scale = 1.0 / (math.sqrt(17.0) * 12.0 * 32.0)
    wnw0_ref[...] = lax.dot_general(node_mat, ww1a_ref[...],
                                    (((0,), (1,)), ((), ()))) * scale
    emb2_ref[...] = ce_ref[...] @ ((ww1b_ref[...] @ node_mat) * scale)


def _node_pass(partials, spe_col, rad_p, hard_p, ce_p, ww1a, ww1b, wx1b, wx2, wx3):
    return pl.pallas_call(
        _node_body,
        out_shape=[
            jax.ShapeDtypeStruct((NN_PAD, 1), jnp.float32),
            jax.ShapeDtypeStruct((1, 1), jnp.float32),
            jax.ShapeDtypeStruct((1, 1), jnp.float32),
            jax.ShapeDtypeStruct((128, MLP_H), jnp.float32),
            jax.ShapeDtypeStruct((MLP_H, 1), jnp.float32),
            jax.ShapeDtypeStruct((1, 128), jnp.float32),
        ],
    )(partials, spe_col, rad_p, hard_p, ce_p, ww1a, ww1b, wx1b, wx2, wx3)


def _sc_gather_body(part_hbm, spec_hbm, sends_hbm, che_hbm, spe_hbm,
                    idx_v, chg_v, spg_v, stg_f, stg_f2, stg_i, ch_sh, sp_sh, sem):
    cid = lax.axis_index("c")
    sid = lax.axis_index("s")
    off = sid * NODES_PER_TILE
    pltpu.sync_copy(part_hbm.at[0, pl.ds(off, NODES_PER_TILE)], stg_f)
    pltpu.sync_copy(part_hbm.at[1, pl.ds(off, NODES_PER_TILE)], stg_f2)

    def _ab(i, carry):
        sl = pl.ds(i * 16, 16)
        stg_f[sl] = stg_f[sl] + stg_f2[sl]
        return carry

    lax.fori_loop(0, NODES_PER_TILE // 16, _ab, 0)
    pltpu.sync_copy(stg_f, ch_sh.at[pl.ds(off, NODES_PER_TILE)])
    pltpu.sync_copy(spec_hbm.at[pl.ds(off, NODES_PER_TILE)], stg_i)
    pltpu.sync_copy(stg_i, sp_sh.at[pl.ds(off, NODES_PER_TILE)])
    plsc.subcore_barrier()

    base = (cid * 16 + sid) * E_PER_W
    pltpu.sync_copy(sends_hbm.at[pl.ds(base, E_PER_W)], idx_v)
    pltpu.async_copy(ch_sh.at[idx_v], chg_v, sem).wait()
    pltpu.async_copy(sp_sh.at[idx_v], spg_v, sem).wait()
    pltpu.sync_copy(chg_v, che_hbm.at[pl.ds(base, E_PER_W)])
    pltpu.sync_copy(spg_v, spe_hbm.at[pl.ds(base, E_PER_W)])


def _sc_gather(partials, species_tab, senders):
    mesh = plsc.VectorSubcoreMesh(core_axis_name="c", subcore_axis_name="s")
    f = functools.partial(
        pl.kernel,
        out_type=(
            jax.ShapeDtypeStruct((N_EDGES,), jnp.float32),
            jax.ShapeDtypeStruct((N_EDGES,), jnp.int32),
        ),
        mesh=mesh,
        scratch_types=[
            pltpu.VMEM((E_PER_W,), jnp.int32),
            pltpu.VMEM((E_PER_W,), jnp.float32),
            pltpu.VMEM((E_PER_W,), jnp.int32),
            pltpu.VMEM((NODES_PER_TILE,), jnp.float32),
            pltpu.VMEM((NODES_PER_TILE,), jnp.float32),
            pltpu.VMEM((NODES_PER_TILE,), jnp.int32),
            pltpu.VMEM_SHARED((NN_PAD,), jnp.float32),
            pltpu.VMEM_SHARED((NN_PAD,), jnp.int32),
            pltpu.SemaphoreType.DMA,
        ],
    )(_sc_gather_body)
    return f(partials, species_tab, senders)


_C1 = (P + 1) * (P + 2) / 2.0
_C2 = P * (P + 2)
_C3 = P * (P + 1) / 2.0


def _pass2_body(at_ref, v_ref, chg_ref, spe_ref, mean_ref,
                emb2_ref, wnw0_ref, diags_ref, o_ref):
    a_t = at_ref[...].astype(jnp.float32)
    vt = v_ref[...]
    u2 = jnp.sum(vt * vt, axis=0, keepdims=True)
    u = jnp.sqrt(u2)
    u6 = u2 * u2 * u2
    env = 1.0 - _C1 * u6 + _C2 * (u6 * u) - _C3 * (u6 * u2)
    env = jnp.where(u < 1.0, env, 0.0)

    spe_row = spe_ref[...].reshape(1, E2)
    lane_col = lax.broadcasted_iota(jnp.int32, (128, 1), 0)
    oh_t = (lane_col == spe_row).astype(jnp.float32)

    emb_t = lax.dot_general(emb2_ref[...], oh_t, (((0,), (0,)), ((), ())),
                            preferred_element_type=jnp.float32)
    diag_row = lax.dot_general(diags_ref[...], oh_t, (((1,), (0,)), ((), ())),
                               preferred_element_type=jnp.float32)

    chg_row = chg_ref[...].reshape(1, E2)
    ch_row = -chg_row / (diag_row + EPS) - mean_ref[...]
    h_t = a_t + wnw0_ref[...] * ch_row + emb_t
    o_ref[...] = env * h_t


def _pass2(a_t, vec_t, chg3, spe3, mean, emb2, wnw0, diag_s):
    nb = N_EDGES // E2
    return pl.pallas_call(
        _pass2_body,
        grid=(nb,),
        in_specs=[
            pl.BlockSpec((MLP_H, E2), lambda i: (0, i)),
            pl.BlockSpec((3, E2), lambda i: (0, i)),
            pl.BlockSpec((1, 1, E2), lambda i: (i, 0, 0)),
            pl.BlockSpec((1, 1, E2), lambda i: (i, 0, 0)),
            pl.BlockSpec((1, 1), lambda i: (0, 0)),
            pl.BlockSpec((128, MLP_H), lambda i: (0, 0)),
            pl.BlockSpec((MLP_H, 1), lambda i: (0, 0)),
            pl.BlockSpec((1, 128), lambda i: (0, 0)),
        ],
        out_specs=pl.BlockSpec((MLP_H, E2), lambda i: (0, i)),
        out_shape=jax.ShapeDtypeStruct((MLP_H, N_EDGES), jnp.float32),
    )(a_t, vec_t, chg3, spe3, mean, emb2, wnw0, diag_s)


def kernel(vectors, x, V, senders, species, radius, hardness, charge_embed,
           W_chi1, W_chi2, W_w1, W_x1, W_x2, W_x3):
    wx1a = W_x1[:D_FEAT]
    wx1b = W_x1[D_FEAT:]
    ww1a = W_w1[0:1]
    ww1b = W_w1[1:]

    a_t, chis3 = _pass1(x, wx1a, W_x2, W_x3, W_chi1, W_chi2)

    partials = _sc_scatter(senders, chis3.reshape(N_EDGES))

    spe_col = jnp.pad(species, (0, NN_PAD - N_NODES)).reshape(NN_PAD, 1)
    rad_p = jnp.pad(radius, (0, 128 - radius.shape[0])).reshape(1, 128)
    hard_p = jnp.pad(hardness, (0, 128 - hardness.shape[0])).reshape(1, 128)
    ce_p = jnp.pad(charge_embed, ((0, 128 - charge_embed.shape[0]), (0, 0)))

    charges_col, pot, mean, emb2, wnw0, diag_s = _node_pass(
        partials, spe_col, rad_p, hard_p, ce_p, ww1a, ww1b, wx1b, W_x2, W_x3)

    spe_pad = jnp.pad(species, (0, NN_PAD - N_NODES))
    chg_e, spe_e = _sc_gather(partials, spe_pad, senders)

    nb = N_EDGES // E2
    o_t = _pass2(a_t, vectors.T, chg_e.reshape(nb, 1, E2),
                 spe_e.reshape(nb, 1, E2), mean, emb2, wnw0, diag_s)
    x_out = o_t.T

    charges = charges_col.reshape(NN_PAD)[:N_NODES]
    return (x_out, V, charges, pot.reshape(()))


# speedup vs baseline: 1.0090x; 1.0090x over previous
import functools
import math

import jax
import jax.numpy as jnp
from jax import lax
from jax.experimental import pallas as pl
from jax.experimental.pallas import tpu as pltpu
from jax.experimental.pallas import tpu_sc as plsc

N_NODES = 10000
N_EDGES = 320000
D_FEAT = 128
CED = 16
MLP_H = 32
EPS = 1e-06
P = 6

NN_PAD = 10240
NW = 32
E_PER_W = N_EDGES // NW
NODES_PER_TILE = NN_PAD // 16

E1 = 32000
E2 = 32000

_CC = (((0,), (1,)), ((), ()))


def _pass1_body(x_ref, wx1a_ref, wx2_ref, wx3_ref, wc1_ref, wc2_ref,
                at_ref, chis_ref):
    x = x_ref[...]
    w_edge = (wx1a_ref[...] @ wx2_ref[...]) @ wx3_ref[...]
    w_edge = w_edge * (1.0 / (12.0 * 32.0))
    at_ref[...] = lax.dot_general(w_edge, x, _CC,
                                  preferred_element_type=jnp.float32
                                  ).astype(jnp.bfloat16)
    wct = lax.dot_general(wc2_ref[...], wc1_ref[...], _CC)
    wct = wct * (1.0 / math.sqrt(128.0 * 16.0))
    chis = lax.dot_general(wct, x, (((1,), (1,)), ((), ())),
                           preferred_element_type=jnp.float32)
    chis_ref[...] = chis.reshape(1, 1, E1)


def _pass1(x, wx1a, wx2, wx3, wc1, wc2):
    nb = N_EDGES // E1
    return pl.pallas_call(
        _pass1_body,
        grid=(nb,),
        in_specs=[
            pl.BlockSpec((E1, D_FEAT), lambda i: (i, 0)),
            pl.BlockSpec((D_FEAT, MLP_H), lambda i: (0, 0)),
            pl.BlockSpec((MLP_H, MLP_H), lambda i: (0, 0)),
            pl.BlockSpec((MLP_H, MLP_H), lambda i: (0, 0)),
            pl.BlockSpec((D_FEAT, CED), lambda i: (0, 0)),
            pl.BlockSpec((CED, 1), lambda i: (0, 0)),
        ],
        out_specs=[
            pl.BlockSpec((MLP_H, E1), lambda i: (0, i)),
            pl.BlockSpec((1, 1, E1), lambda i: (i, 0, 0)),
        ],
        out_shape=[
            jax.ShapeDtypeStruct((MLP_H, N_EDGES), jnp.bfloat16),
            jax.ShapeDtypeStruct((N_EDGES // E1, 1, E1), jnp.float32),
        ],
    )(x, wx1a, wx2, wx3, wc1, wc2)


CHUNK = 2000
CH_PER_W = E_PER_W // CHUNK
CH_PER_ROW1 = E1 // CHUNK
CH_PER_ROW2 = E2 // CHUNK


def _sc_scatter_body(sends_hbm, vals_hbm, out_hbm, idx_v, val_v, zero_v, acc_sh):
    cid = lax.axis_index("c")
    sid = lax.axis_index("s")
    z16 = jnp.zeros((16,), jnp.float32)

    def _zb(i, carry):
        zero_v[pl.ds(i * 16, 16)] = z16
        return carry

    lax.fori_loop(0, NODES_PER_TILE // 16, _zb, 0)
    pltpu.sync_copy(zero_v, acc_sh.at[pl.ds(sid * NODES_PER_TILE, NODES_PER_TILE)])
    plsc.subcore_barrier()

    base = (cid * 16 + sid) * E_PER_W
    pltpu.sync_copy(sends_hbm.at[pl.ds(base, E_PER_W)], idx_v)
    pltpu.sync_copy(vals_hbm.at[pl.ds(base, E_PER_W)], val_v)
    pltpu.sync_copy(val_v, acc_sh.at[idx_v], add=True)
    plsc.subcore_barrier()

    pltpu.sync_copy(acc_sh.at[pl.ds(sid * NODES_PER_TILE, NODES_PER_TILE)],
                    out_hbm.at[cid, pl.ds(sid * NODES_PER_TILE, NODES_PER_TILE)])


def _sc_scatter(senders, chis_flat):
    mesh = plsc.VectorSubcoreMesh(core_axis_name="c", subcore_axis_name="s")
    f = functools.partial(
        pl.kernel,
        out_type=jax.ShapeDtypeStruct((2, NN_PAD), jnp.float32),
        mesh=mesh,
        scratch_types=[
            pltpu.VMEM((E_PER_W,), jnp.int32),
            pltpu.VMEM((E_PER_W,), jnp.float32),
            pltpu.VMEM((NODES_PER_TILE,), jnp.float32),
            pltpu.VMEM_SHARED((NN_PAD,), jnp.float32),
        ],
    )(_sc_scatter_body)
    return f(senders, chis_flat)


def _node_body(p_ref, spe_ref, rad_ref, hard_ref, ce_ref,
               ww1a_ref, ww1b_ref, wx1b_ref, wx2_ref, wx3_ref,
               charges_ref, pot_ref, mean_ref, emb2_ref, wnw0_ref, diags_ref):
    ones21 = jnp.ones((2, 1), jnp.float32)
    chis = lax.dot_general(p_ref[...], ones21, (((0,), (0,)), ((), ())),
                           preferred_element_type=jnp.float32)
    spe = spe_ref[...]
    lane = lax.broadcasted_iota(jnp.int32, (1, 128), 1)
    oh = (spe == lane).astype(jnp.float32)

    hard = hard_ref[...]
    sp = jnp.maximum(hard, 0.0) + jnp.log1p(jnp.exp(-jnp.abs(hard)))
    gam = rad_ref[...] * 4.0 + 0.5
    diag_s = sp + 2.0 / (gam * math.sqrt(math.pi))

    diag = lax.dot_general(oh, diag_s, (((1,), (1,)), ((), ())),
                           preferred_element_type=jnp.float32)
    charges_raw = -chis / (diag + EPS)
    ridx = lax.broadcasted_iota(jnp.int32, (NN_PAD, 1), 0)
    mask = (ridx < N_NODES).astype(jnp.float32)
    charges_raw = charges_raw * mask
    mean = jnp.sum(charges_raw) / N_NODES
    charges = charges_raw - mean
    charges_ref[...] = charges
    pot_ref[...] = jnp.sum(
        mask * (chis * charges + 0.5 * diag * charges * charges)
    ).reshape(1, 1)
    mean_ref[...] = mean.reshape(1, 1)
    diags_ref[...] = diag_s
    node_mat = (wx1b_ref[...] @ wx2_ref[...]) @ wx3_ref[...]
    scale = 1.0 / (math.sqrt(17.0) * 12.0 * 32.0)
    wnw0_ref[...] = lax.dot_general(node_mat, ww1a_ref[...],
                                    (((0,), (1,)), ((), ()))) * scale
    emb2_ref[...] = ce_ref[...] @ ((ww1b_ref[...] @ node_mat) * scale)


def _node_pass(partials, spe_col, rad_p, hard_p, ce_p, ww1a, ww1b, wx1b, wx2, wx3):
    return pl.pallas_call(
        _node_body,
        out_shape=[
            jax.ShapeDtypeStruct((NN_PAD, 1), jnp.float32),
            jax.ShapeDtypeStruct((1, 1), jnp.float32),
            jax.ShapeDtypeStruct((1, 1), jnp.float32),
            jax.ShapeDtypeStruct((128, MLP_H), jnp.float32),
            jax.ShapeDtypeStruct((MLP_H, 1), jnp.float32),
            jax.ShapeDtypeStruct((1, 128), jnp.float32),
        ],
    )(partials, spe_col, rad_p, hard_p, ce_p, ww1a, ww1b, wx1b, wx2, wx3)


def _sc_gather_body(part_hbm, spec_hbm, sends_hbm, che_hbm, spe_hbm,
                    idx_v, chg_v, spg_v, stg_f, stg_f2, stg_i, ch_sh, sp_sh, sem):
    cid = lax.axis_index("c")
    sid = lax.axis_index("s")
    off = sid * NODES_PER_TILE
    pltpu.sync_copy(part_hbm.at[0, pl.ds(off, NODES_PER_TILE)], stg_f)
    pltpu.sync_copy(part_hbm.at[1, pl.ds(off, NODES_PER_TILE)], stg_f2)

    def _ab(i, carry):
        sl = pl.ds(i * 16, 16)
        stg_f[sl] = stg_f[sl] + stg_f2[sl]
        return carry

    lax.fori_loop(0, NODES_PER_TILE // 16, _ab, 0)
    pltpu.sync_copy(stg_f, ch_sh.at[pl.ds(off, NODES_PER_TILE)])
    pltpu.sync_copy(spec_hbm.at[pl.ds(off, NODES_PER_TILE)], stg_i)
    pltpu.sync_copy(stg_i, sp_sh.at[pl.ds(off, NODES_PER_TILE)])
    plsc.subcore_barrier()

    base = (cid * 16 + sid) * E_PER_W
    pltpu.sync_copy(sends_hbm.at[pl.ds(base, E_PER_W)], idx_v)
    pltpu.async_copy(ch_sh.at[idx_v], chg_v, sem).wait()
    pltpu.async_copy(sp_sh.at[idx_v], spg_v, sem).wait()
    pltpu.sync_copy(chg_v, che_hbm.at[pl.ds(base, E_PER_W)])
    pltpu.sync_copy(spg_v, spe_hbm.at[pl.ds(base, E_PER_W)])


def _sc_gather(partials, species_tab, senders):
    mesh = plsc.VectorSubcoreMesh(core_axis_name="c", subcore_axis_name="s")
    f = functools.partial(
        pl.kernel,
        out_type=(
            jax.ShapeDtypeStruct((N_EDGES,), jnp.float32),
            jax.ShapeDtypeStruct((N_EDGES,), jnp.int32),
        ),
        mesh=mesh,
        scratch_types=[
            pltpu.VMEM((E_PER_W,), jnp.int32),
            pltpu.VMEM((E_PER_W,), jnp.float32),
            pltpu.VMEM((E_PER_W,), jnp.int32),
            pltpu.VMEM((NODES_PER_TILE,), jnp.float32),
            pltpu.VMEM((NODES_PER_TILE,), jnp.float32),
            pltpu.VMEM((NODES_PER_TILE,), jnp.int32),
            pltpu.VMEM_SHARED((NN_PAD,), jnp.float32),
            pltpu.VMEM_SHARED((NN_PAD,), jnp.int32),
            pltpu.SemaphoreType.DMA,
        ],
    )(_sc_gather_body)
    return f(partials, species_tab, senders)


_C1 = (P + 1) * (P + 2) / 2.0
_C2 = P * (P + 2)
_C3 = P * (P + 1) / 2.0


def _pass2_body(at_ref, v_ref, chg_ref, spe_ref, mean_ref,
                emb2_ref, wnw0_ref, diags_ref, o_ref):
    a_t = at_ref[...].astype(jnp.float32)
    vt = v_ref[...]
    u2 = jnp.sum(vt * vt, axis=0, keepdims=True)
    u = jnp.sqrt(u2)
    u6 = u2 * u2 * u2
    env = 1.0 - _C1 * u6 + _C2 * (u6 * u) - _C3 * (u6 * u2)
    env = jnp.where(u < 1.0, env, 0.0)

    spe_row = spe_ref[...].reshape(1, E2)
    lane_col = lax.broadcasted_iota(jnp.int32, (128, 1), 0)
    oh_t = (lane_col == spe_row).astype(jnp.float32)

    emb_t = lax.dot_general(emb2_ref[...], oh_t, (((0,), (0,)), ((), ())),
                            preferred_element_type=jnp.float32)
    diag_row = lax.dot_general(diags_ref[...], oh_t, (((1,), (0,)), ((), ())),
                               preferred_element_type=jnp.float32)

    chg_row = chg_ref[...].reshape(1, E2)
    ch_row = -chg_row / (diag_row + EPS) - mean_ref[...]
    h_t = a_t + wnw0_ref[...] * ch_row + emb_t
    o_ref[...] = env * h_t


def _pass2(a_t, vec_t, chg3, spe3, mean, emb2, wnw0, diag_s):
    nb = N_EDGES // E2
    return pl.pallas_call(
        _pass2_body,
        grid=(nb,),
        in_specs=[
            pl.BlockSpec((MLP_H, E2), lambda i: (0, i)),
            pl.BlockSpec((3, E2), lambda i: (0, i)),
            pl.BlockSpec((1, 1, E2), lambda i: (i, 0, 0)),
            pl.BlockSpec((1, 1, E2), lambda i: (i, 0, 0)),
            pl.BlockSpec((1, 1), lambda i: (0, 0)),
            pl.BlockSpec((128, MLP_H), lambda i: (0, 0)),
            pl.BlockSpec((MLP_H, 1), lambda i: (0, 0)),
            pl.BlockSpec((1, 128), lambda i: (0, 0)),
        ],
        out_specs=pl.BlockSpec((MLP_H, E2), lambda i: (0, i)),
        out_shape=jax.ShapeDtypeStruct((MLP_H, N_EDGES), jnp.float32),
    )(a_t, vec_t, chg3, spe3, mean, emb2, wnw0, diag_s)


def kernel(vectors, x, V, senders, species, radius, hardness, charge_embed,
           W_chi1, W_chi2, W_w1, W_x1, W_x2, W_x3):
    wx1a = W_x1[:D_FEAT]
    wx1b = W_x1[D_FEAT:]
    ww1a = W_w1[0:1]
    ww1b = W_w1[1:]

    a_t, chis3 = _pass1(x, wx1a, W_x2, W_x3, W_chi1, W_chi2)

    partials = _sc_scatter(senders, chis3.reshape(N_EDGES))

    spe_col = jnp.pad(species, (0, NN_PAD - N_NODES)).reshape(NN_PAD, 1)
    rad_p = jnp.pad(radius, (0, 128 - radius.shape[0])).reshape(1, 128)
    hard_p = jnp.pad(hardness, (0, 128 - hardness.shape[0])).reshape(1, 128)
    ce_p = jnp.pad(charge_embed, ((0, 128 - charge_embed.shape[0]), (0, 0)))

    charges_col, pot, mean, emb2, wnw0, diag_s = _node_pass(
        partials, spe_col, rad_p, hard_p, ce_p, ww1a, ww1b, wx1b, W_x2, W_x3)

    spe_pad = jnp.pad(species, (0, NN_PAD - N_NODES))
    chg_e, spe_e = _sc_gather(partials, spe_pad, senders)

    nb = N_EDGES // E2
    o_t = _pass2(a_t, vectors.T, chg_e.reshape(nb, 1, E2),
                 spe_e.reshape(nb, 1, E2), mean, emb2, wnw0, diag_s)
    x_out = o_t.T

    charges = charges_col.reshape(NN_PAD)[:N_NODES]
    return (x_out, V, charges, pot.reshape(()))


# speedup vs baseline: 1.0094x; 1.0004x over previous
import functools
import math

import jax
import jax.numpy as jnp
from jax import lax
from jax.experimental import pallas as pl
from jax.experimental.pallas import tpu as pltpu
from jax.experimental.pallas import tpu_sc as plsc

N_NODES = 10000
N_EDGES = 320000
D_FEAT = 128
CED = 16
MLP_H = 32
EPS = 1e-06
P = 6

NN_PAD = 10240
NW = 32
E_PER_W = N_EDGES // NW
NODES_PER_TILE = NN_PAD // 16

E1 = 32000
E2 = 32000

_CC = (((0,), (1,)), ((), ()))


def _pass1_body(x_ref, wx1a_ref, wx2_ref, wx3_ref, wc1_ref, wc2_ref,
                at_ref, chis_ref):
    x = x_ref[...]
    w_edge = (wx1a_ref[...] @ wx2_ref[...]) @ wx3_ref[...]
    w_edge = w_edge * (1.0 / (12.0 * 32.0))
    at_ref[...] = lax.dot_general(w_edge, x, _CC,
                                  preferred_element_type=jnp.float32
                                  ).astype(jnp.bfloat16)
    wct = lax.dot_general(wc2_ref[...], wc1_ref[...], _CC)
    wct = wct * (1.0 / math.sqrt(128.0 * 16.0))
    chis = lax.dot_general(wct, x, (((1,), (1,)), ((), ())),
                           preferred_element_type=jnp.float32)
    chis_ref[...] = chis.reshape(1, 1, E1)


def _pass1(x, wx1a, wx2, wx3, wc1, wc2):
    nb = N_EDGES // E1
    return pl.pallas_call(
        _pass1_body,
        grid=(nb,),
        in_specs=[
            pl.BlockSpec((E1, D_FEAT), lambda i: (i, 0)),
            pl.BlockSpec((D_FEAT, MLP_H), lambda i: (0, 0)),
            pl.BlockSpec((MLP_H, MLP_H), lambda i: (0, 0)),
            pl.BlockSpec((MLP_H, MLP_H), lambda i: (0, 0)),
            pl.BlockSpec((D_FEAT, CED), lambda i: (0, 0)),
            pl.BlockSpec((CED, 1), lambda i: (0, 0)),
        ],
        out_specs=[
            pl.BlockSpec((MLP_H, E1), lambda i: (0, i)),
            pl.BlockSpec((1, 1, E1), lambda i: (i, 0, 0)),
        ],
        out_shape=[
            jax.ShapeDtypeStruct((MLP_H, N_EDGES), jnp.bfloat16),
            jax.ShapeDtypeStruct((N_EDGES // E1, 1, E1), jnp.float32),
        ],
    )(x, wx1a, wx2, wx3, wc1, wc2)


def _sc_scatter_body(sends_hbm, vals_hbm, out_hbm, idx_v, val_v, zero_v, acc_sh):
    cid = lax.axis_index("c")
    sid = lax.axis_index("s")
    z16 = jnp.zeros((16,), jnp.float32)

    def _zb(i, carry):
        zero_v[pl.ds(i * 16, 16)] = z16
        return carry

    lax.fori_loop(0, NODES_PER_TILE // 16, _zb, 0)
    pltpu.sync_copy(zero_v, acc_sh.at[pl.ds(sid * NODES_PER_TILE, NODES_PER_TILE)])
    plsc.subcore_barrier()

    base = (cid * 16 + sid) * E_PER_W
    pltpu.sync_copy(sends_hbm.at[pl.ds(base, E_PER_W)], idx_v)
    pltpu.sync_copy(vals_hbm.at[pl.ds(base, E_PER_W)], val_v)
    pltpu.sync_copy(val_v, acc_sh.at[idx_v], add=True)
    plsc.subcore_barrier()

    pltpu.sync_copy(acc_sh.at[pl.ds(sid * NODES_PER_TILE, NODES_PER_TILE)],
                    out_hbm.at[cid, pl.ds(sid * NODES_PER_TILE, NODES_PER_TILE)])


def _sc_scatter(senders, chis_flat):
    mesh = plsc.VectorSubcoreMesh(core_axis_name="c", subcore_axis_name="s")
    f = functools.partial(
        pl.kernel,
        out_type=jax.ShapeDtypeStruct((2, NN_PAD), jnp.float32),
        mesh=mesh,
        scratch_types=[
            pltpu.VMEM((E_PER_W,), jnp.int32),
            pltpu.VMEM((E_PER_W,), jnp.float32),
            pltpu.VMEM((NODES_PER_TILE,), jnp.float32),
            pltpu.VMEM_SHARED((NN_PAD,), jnp.float32),
        ],
    )(_sc_scatter_body)
    return f(senders, chis_flat)


def _node_body(p_ref, spe_ref, rad_ref, hard_ref, ce_ref,
               ww1a_ref, ww1b_ref, wx1b_ref, wx2_ref, wx3_ref,
               charges_ref, pot_ref, mean_ref, emb2_ref, wnw0_ref, diags_ref):
    ones21 = jnp.ones((2, 1), jnp.float32)
    chis = lax.dot_general(p_ref[...], ones21, (((0,), (0,)), ((), ())),
                           preferred_element_type=jnp.float32)
    spe = spe_ref[...]
    lane = lax.broadcasted_iota(jnp.int32, (1, 128), 1)
    oh = (spe == lane).astype(jnp.float32)

    hard = hard_ref[...]
    sp = jnp.maximum(hard, 0.0) + jnp.log1p(jnp.exp(-jnp.abs(hard)))
    gam = rad_ref[...] * 4.0 + 0.5
    diag_s = sp + 2.0 / (gam * math.sqrt(math.pi))

    diag = lax.dot_general(oh, diag_s, (((1,), (1,)), ((), ())),
                           preferred_element_type=jnp.float32)
    charges_raw = -chis / (diag + EPS)
    ridx = lax.broadcasted_iota(jnp.int32, (NN_PAD, 1), 0)
    mask = (ridx < N_NODES).astype(jnp.float32)
    charges_raw = charges_raw * mask
    mean = jnp.sum(charges_raw) / N_NODES
    charges = charges_raw - mean
    charges_ref[...] = charges
    pot_ref[...] = jnp.sum(
        mask * (chis * charges + 0.5 * diag * charges * charges)
    ).reshape(1, 1)
    mean_ref[...] = mean.reshape(1, 1)
    diags_ref[...] = diag_s
    node_mat = (wx1b_ref[...] @ wx2_ref[...]) @ wx3_ref[...]
    scale = 1.0 / (math.sqrt(17.0) * 12.0 * 32.0)
    wnw0_ref[...] = lax.dot_general(node_mat, ww1a_ref[...],
                                    (((0,), (1,)), ((), ()))) * scale
    emb2_ref[...] = ce_ref[...] @ ((ww1b_ref[...] @ node_mat) * scale)


def _node_pass(partials, spe_col, rad_p, hard_p, ce_p, ww1a, ww1b, wx1b, wx2, wx3):
    return pl.pallas_call(
        _node_body,
        out_shape=[
            jax.ShapeDtypeStruct((NN_PAD, 1), jnp.float32),
            jax.ShapeDtypeStruct((1, 1), jnp.float32),
            jax.ShapeDtypeStruct((1, 1), jnp.float32),
            jax.ShapeDtypeStruct((128, MLP_H), jnp.float32),
            jax.ShapeDtypeStruct((MLP_H, 1), jnp.float32),
            jax.ShapeDtypeStruct((1, 128), jnp.float32),
        ],
    )(partials, spe_col, rad_p, hard_p, ce_p, ww1a, ww1b, wx1b, wx2, wx3)


def _sc_gather_body(part_hbm, spec_hbm, sends_hbm, che_hbm, spe_hbm,
                    idx_v, chg_v, spg_v, stg_f, stg_f2, stg_i, ch_sh, sp_sh, sem):
    cid = lax.axis_index("c")
    sid = lax.axis_index("s")
    off = sid * NODES_PER_TILE
    pltpu.sync_copy(part_hbm.at[0, pl.ds(off, NODES_PER_TILE)], stg_f)
    pltpu.sync_copy(part_hbm.at[1, pl.ds(off, NODES_PER_TILE)], stg_f2)

    def _ab(i, carry):
        sl = pl.ds(i * 16, 16)
        stg_f[sl] = stg_f[sl] + stg_f2[sl]
        return carry

    lax.fori_loop(0, NODES_PER_TILE // 16, _ab, 0)
    pltpu.sync_copy(stg_f, ch_sh.at[pl.ds(off, NODES_PER_TILE)])
    pltpu.sync_copy(spec_hbm.at[pl.ds(off, NODES_PER_TILE)], stg_i)
    pltpu.sync_copy(stg_i, sp_sh.at[pl.ds(off, NODES_PER_TILE)])
    plsc.subcore_barrier()

    base = (cid * 16 + sid) * E_PER_W
    pltpu.sync_copy(sends_hbm.at[pl.ds(base, E_PER_W)], idx_v)
    pltpu.async_copy(ch_sh.at[idx_v], chg_v, sem).wait()
    pltpu.async_copy(sp_sh.at[idx_v], spg_v, sem).wait()
    pltpu.sync_copy(chg_v, che_hbm.at[pl.ds(base, E_PER_W)])
    pltpu.sync_copy(spg_v, spe_hbm.at[pl.ds(base, E_PER_W)])


def _sc_gather(partials, species_tab, senders):
    mesh = plsc.VectorSubcoreMesh(core_axis_name="c", subcore_axis_name="s")
    f = functools.partial(
        pl.kernel,
        out_type=(
            jax.ShapeDtypeStruct((N_EDGES,), jnp.float32),
            jax.ShapeDtypeStruct((N_EDGES,), jnp.int32),
        ),
        mesh=mesh,
        scratch_types=[
            pltpu.VMEM((E_PER_W,), jnp.int32),
            pltpu.VMEM((E_PER_W,), jnp.float32),
            pltpu.VMEM((E_PER_W,), jnp.int32),
            pltpu.VMEM((NODES_PER_TILE,), jnp.float32),
            pltpu.VMEM((NODES_PER_TILE,), jnp.float32),
            pltpu.VMEM((NODES_PER_TILE,), jnp.int32),
            pltpu.VMEM_SHARED((NN_PAD,), jnp.float32),
            pltpu.VMEM_SHARED((NN_PAD,), jnp.int32),
            pltpu.SemaphoreType.DMA,
        ],
    )(_sc_gather_body)
    return f(partials, species_tab, senders)


_C1 = (P + 1) * (P + 2) / 2.0
_C2 = P * (P + 2)
_C3 = P * (P + 1) / 2.0


def _pass2_body(at_ref, v_ref, chg_ref, spe_ref, mean_ref,
                emb2_ref, wnw0_ref, diags_ref, o_ref):
    a_t = at_ref[...].astype(jnp.float32)
    vt = v_ref[...]
    u2 = jnp.sum(vt * vt, axis=0, keepdims=True)
    u = jnp.sqrt(u2)
    u6 = u2 * u2 * u2
    env = 1.0 - _C1 * u6 + _C2 * (u6 * u) - _C3 * (u6 * u2)
    env = jnp.where(u < 1.0, env, 0.0)

    spe_row = spe_ref[...].reshape(1, E2)
    lane_col = lax.broadcasted_iota(jnp.int32, (128, 1), 0)
    oh_t = (lane_col == spe_row).astype(jnp.float32)

    emb_t = lax.dot_general(emb2_ref[...], oh_t, (((0,), (0,)), ((), ())),
                            preferred_element_type=jnp.float32)
    diag_row = lax.dot_general(diags_ref[...], oh_t, (((1,), (0,)), ((), ())),
                               preferred_element_type=jnp.float32)

    chg_row = chg_ref[...].reshape(1, E2)
    ch_row = -chg_row / (diag_row + EPS) - mean_ref[...]
    h_t = a_t + wnw0_ref[...] * ch_row + emb_t
    o_ref[...] = env * h_t


def _pass2(a_t, vec_t, chg3, spe3, mean, emb2, wnw0, diag_s):
    nb = N_EDGES // E2
    return pl.pallas_call(
        _pass2_body,
        grid=(nb,),
        in_specs=[
            pl.BlockSpec((MLP_H, E2), lambda i: (0, i)),
            pl.BlockSpec((3, E2), lambda i: (0, i)),
            pl.BlockSpec((1, 1, E2), lambda i: (i, 0, 0)),
            pl.BlockSpec((1, 1, E2), lambda i: (i, 0, 0)),
            pl.BlockSpec((1, 1), lambda i: (0, 0)),
            pl.BlockSpec((128, MLP_H), lambda i: (0, 0)),
            pl.BlockSpec((MLP_H, 1), lambda i: (0, 0)),
            pl.BlockSpec((1, 128), lambda i: (0, 0)),
        ],
        out_specs=pl.BlockSpec((MLP_H, E2), lambda i: (0, i)),
        out_shape=jax.ShapeDtypeStruct((MLP_H, N_EDGES), jnp.float32),
    )(a_t, vec_t, chg3, spe3, mean, emb2, wnw0, diag_s)


def kernel(vectors, x, V, senders, species, radius, hardness, charge_embed,
           W_chi1, W_chi2, W_w1, W_x1, W_x2, W_x3):
    wx1a = W_x1[:D_FEAT]
    wx1b = W_x1[D_FEAT:]
    ww1a = W_w1[0:1]
    ww1b = W_w1[1:]

    a_t, chis3 = _pass1(x, wx1a, W_x2, W_x3, W_chi1, W_chi2)

    partials = _sc_scatter(senders, chis3.reshape(N_EDGES))

    spe_col = jnp.pad(species, (0, NN_PAD - N_NODES)).reshape(NN_PAD, 1)
    rad_p = jnp.pad(radius, (0, 128 - radius.shape[0])).reshape(1, 128)
    hard_p = jnp.pad(hardness, (0, 128 - hardness.shape[0])).reshape(1, 128)
    ce_p = jnp.pad(charge_embed, ((0, 128 - charge_embed.shape[0]), (0, 0)))

    charges_col, pot, mean, emb2, wnw0, diag_s = _node_pass(
        partials, spe_col, rad_p, hard_p, ce_p, ww1a, ww1b, wx1b, W_x2, W_x3)

    spe_pad = jnp.pad(species, (0, NN_PAD - N_NODES))
    chg_e, spe_e = _sc_gather(partials, spe_pad, senders)

    nb = N_EDGES // E2
    o_t = _pass2(a_t, vectors.T, chg_e.reshape(nb, 1, E2),
                 spe_e.reshape(nb, 1, E2), mean, emb2, wnw0, diag_s)
    x_out = o_t.T

    charges = charges_col.reshape(NN_PAD)[:N_NODES]
    return (x_out, V, charges, pot.reshape(()))
